# baseline (speedup 1.0000x reference)
import jax
import jax.numpy as jnp
from jax.experimental import pallas as pl
from jax.experimental.pallas import tpu as pltpu


def _body(x_ref, t_ref, o_ref, *, scale):
    o_ref[...] = x_ref[...] + t_ref[...] * scale


def kernel(inputs, lookup_table):
    B, T, U = inputs.shape
    scale = float(U) ** 0.5
    D = T * U
    x = inputs.reshape(B, D)
    tab = lookup_table.reshape(1, D)

    BB = 256
    grid = (B // BB,)
    import functools
    out = pl.pallas_call(
        functools.partial(_body, scale=scale),
        grid=grid,
        in_specs=[
            pl.BlockSpec((BB, D), lambda i: (i, 0)),
            pl.BlockSpec((1, D), lambda i: (0, 0)),
        ],
        out_specs=pl.BlockSpec((BB, D), lambda i: (i, 0)),
        out_shape=jax.ShapeDtypeStruct((B, D), jnp.float32),
        compiler_params=pltpu.CompilerParams(
            dimension_semantics=("arbitrary",),
        ),
    )(x, tab)
    return out.reshape(B, T, U)
